# baseline (speedup 1.0000x reference)
import jax
import jax.numpy as jnp
from jax import lax
from jax.experimental import pallas as pl
from jax.experimental.pallas import tpu as pltpu
from jax.experimental.pallas import tpu_sc as plsc

SEQ = 200
DIM = 32
VOCAB = 1000000
BATCH = 4096
B_TOTAL = BATCH * SEQ
NC, NS = 2, 16
NW = NC * NS
BLK = 128
N_BLOCKS = B_TOTAL // BLK
BPW = N_BLOCKS // NW
TB = BATCH // BLK
TD = DIM // 8
LANES = 16
NBUF = 8


def _body(idx_hbm, word_hbm, pos_hbm, out_hbm,
          idx_v, pos_v, rows_v, out_t, gsem, osem):
    cid = lax.axis_index("c")
    sid = lax.axis_index("s")
    wid = sid * NC + cid
    blk0 = wid * BPW

    pltpu.sync_copy(idx_hbm.at[pl.ds(blk0, BPW)], idx_v)
    pltpu.sync_copy(pos_hbm, pos_v)

    def gather(blk, buf):
        pltpu.async_copy(word_hbm.at[idx_v.at[blk]], rows_v.at[buf],
                         gsem.at[buf])

    def gather_wait(blk, buf):
        pltpu.make_async_copy(word_hbm.at[idx_v.at[blk]], rows_v.at[buf],
                              gsem.at[buf]).wait()

    def out_write(gblk, buf):
        s = lax.shift_right_logical(gblk, 5)
        tb = lax.bitwise_and(gblk, TB - 1)
        for td in range(TD):
            pltpu.async_copy(out_t.at[buf, pl.ds(td * 8, 8), pl.ds(0, BLK)],
                             out_hbm.at[s, td, tb], osem.at[buf])

    def out_drain(buf):
        for td in range(TD):
            pltpu.make_async_copy(out_t.at[buf, pl.ds(td * 8, 8),
                                           pl.ds(0, BLK)],
                                  out_hbm.at[0, td, 0], osem.at[buf]).wait()

    iota = lax.broadcasted_iota(jnp.int32, (LANES,), 0)
    iota_hi = iota + LANES

    for p in range(NBUF - 1):
        gather(p, p)

    @pl.loop(0, BPW)
    def _blocks(blk):
        buf = lax.bitwise_and(blk, NBUF - 1)

        gather_wait(blk, buf)

        @pl.when(blk >= NBUF)
        def _():
            out_drain(buf)

        gblk = blk0 + blk
        s = lax.shift_right_logical(gblk, 5)
        pv0 = pos_v[s, pl.ds(0, LANES)]
        pv1 = pos_v[s, pl.ds(LANES, LANES)]

        @pl.loop(0, BLK)
        def _b(b):
            bvec = jnp.broadcast_to(b, (LANES,))
            ot = out_t.at[buf]
            plsc.store_scatter(ot, [iota, bvec],
                               rows_v[buf, b, pl.ds(0, LANES)] + pv0)
            plsc.store_scatter(ot, [iota_hi, bvec],
                               rows_v[buf, b, pl.ds(LANES, LANES)] + pv1)

        out_write(gblk, buf)

        @pl.when(blk + NBUF - 1 < BPW)
        def _():
            nblk = blk + NBUF - 1
            gather(nblk, lax.bitwise_and(nblk, NBUF - 1))

    for db in range(NBUF):
        out_drain(jnp.int32(db))


_mesh = plsc.VectorSubcoreMesh(core_axis_name="c", subcore_axis_name="s")

_sc_call = pl.kernel(
    _body,
    out_type=jax.ShapeDtypeStruct((SEQ, TD, TB, 8, BLK), jnp.float32),
    mesh=_mesh,
    scratch_types=[
        pltpu.VMEM((BPW, BLK), jnp.int32),
        pltpu.VMEM((SEQ, DIM), jnp.float32),
        pltpu.VMEM((NBUF, BLK, DIM), jnp.float32),
        pltpu.VMEM((NBUF, DIM, BLK + 1), jnp.float32),
        pltpu.SemaphoreType.DMA((NBUF,)),
        pltpu.SemaphoreType.DMA((NBUF,)),
    ],
    compiler_params=pltpu.CompilerParams(use_tc_tiling_on_sc=False,
                                         needs_layout_passes=False),
)


@jax.jit
def kernel(inputs, word_table, pos_table):
    idx = (inputs.astype(jnp.int32) * 4).T.reshape(N_BLOCKS, BLK)
    word_pad = jnp.pad(word_table, ((0, 0), (0, 96))).reshape(4 * VOCAB, DIM)
    out5 = _sc_call(idx, word_pad, pos_table)
    out = out5.transpose(2, 4, 0, 1, 3).reshape(BATCH, SEQ, DIM)
    return out


# speedup vs baseline: 1.0017x; 1.0017x over previous
import jax
import jax.numpy as jnp
from jax import lax
from jax.experimental import pallas as pl
from jax.experimental.pallas import tpu as pltpu
from jax.experimental.pallas import tpu_sc as plsc

SEQ = 200
DIM = 32
VOCAB = 1000000
BATCH = 4096
B_TOTAL = BATCH * SEQ
NC, NS = 2, 16
NW = NC * NS
BLK = 128
N_BLOCKS = B_TOTAL // BLK
BPW = N_BLOCKS // NW
TB = BATCH // BLK
TD = DIM // 8
LANES = 16
NBUF = 4


def _body(idx_hbm, word_hbm, pos_hbm, out_hbm,
          idx_v, pos_v, rows_v, out_t, gsem, osem):
    cid = lax.axis_index("c")
    sid = lax.axis_index("s")
    wid = sid * NC + cid
    blk0 = wid * BPW

    pltpu.sync_copy(idx_hbm.at[pl.ds(blk0, BPW)], idx_v)
    pltpu.sync_copy(pos_hbm, pos_v)

    def gather(blk, buf):
        pltpu.async_copy(word_hbm.at[idx_v.at[blk]], rows_v.at[buf],
                         gsem.at[buf])

    def gather_wait(blk, buf):
        pltpu.make_async_copy(word_hbm.at[idx_v.at[blk]], rows_v.at[buf],
                              gsem.at[buf]).wait()

    def out_write(gblk, buf):
        s = lax.shift_right_logical(gblk, 5)
        tb = lax.bitwise_and(gblk, TB - 1)
        for td in range(TD):
            pltpu.async_copy(out_t.at[buf, pl.ds(td * 8, 8), pl.ds(0, BLK)],
                             out_hbm.at[s, td, tb], osem.at[buf])

    def out_drain(buf):
        for td in range(TD):
            pltpu.make_async_copy(out_t.at[buf, pl.ds(td * 8, 8),
                                           pl.ds(0, BLK)],
                                  out_hbm.at[0, td, 0], osem.at[buf]).wait()

    iota = lax.broadcasted_iota(jnp.int32, (LANES,), 0)
    iota_hi = iota + LANES

    for p in range(NBUF - 1):
        gather(p, p)

    @pl.loop(0, BPW)
    def _blocks(blk):
        buf = lax.bitwise_and(blk, NBUF - 1)

        gather_wait(blk, buf)

        @pl.when(blk >= NBUF)
        def _():
            out_drain(buf)

        gblk = blk0 + blk
        s = lax.shift_right_logical(gblk, 5)
        pv0 = pos_v[s, pl.ds(0, LANES)]
        pv1 = pos_v[s, pl.ds(LANES, LANES)]

        @pl.loop(0, BLK)
        def _b(b):
            bvec = jnp.broadcast_to(b, (LANES,))
            ot = out_t.at[buf]
            plsc.store_scatter(ot, [iota, bvec],
                               rows_v[buf, b, pl.ds(0, LANES)] + pv0)
            plsc.store_scatter(ot, [iota_hi, bvec],
                               rows_v[buf, b, pl.ds(LANES, LANES)] + pv1)

        out_write(gblk, buf)

        @pl.when(blk + NBUF - 1 < BPW)
        def _():
            nblk = blk + NBUF - 1
            gather(nblk, lax.bitwise_and(nblk, NBUF - 1))

    for db in range(NBUF):
        out_drain(jnp.int32(db))


_mesh = plsc.VectorSubcoreMesh(core_axis_name="c", subcore_axis_name="s")

_sc_call = pl.kernel(
    _body,
    out_type=jax.ShapeDtypeStruct((SEQ, TD, TB, 8, BLK), jnp.float32),
    mesh=_mesh,
    scratch_types=[
        pltpu.VMEM((BPW, BLK), jnp.int32),
        pltpu.VMEM((SEQ, DIM), jnp.float32),
        pltpu.VMEM((NBUF, BLK, DIM), jnp.float32),
        pltpu.VMEM((NBUF, DIM, BLK + 1), jnp.float32),
        pltpu.SemaphoreType.DMA((NBUF,)),
        pltpu.SemaphoreType.DMA((NBUF,)),
    ],
    compiler_params=pltpu.CompilerParams(use_tc_tiling_on_sc=False,
                                         needs_layout_passes=False),
)


@jax.jit
def kernel(inputs, word_table, pos_table):
    idx = (inputs.astype(jnp.int32) * 4).T.reshape(N_BLOCKS, BLK)
    word_pad = jnp.pad(word_table, ((0, 0), (0, 96))).reshape(4 * VOCAB, DIM)
    out5 = _sc_call(idx, word_pad, pos_table)
    out = out5.transpose(2, 4, 0, 1, 3).reshape(BATCH, SEQ, DIM)
    return out
